# baseline (speedup 1.0000x reference)
import functools

import jax
import jax.numpy as jnp
import numpy as np
from jax import lax
from jax.experimental import pallas as pl
from jax.experimental.pallas import tpu as pltpu
from jax.experimental.pallas import tpu_sc as plsc

B, H, S = 16, 64, 4096
KV_LORA, ROPE = 512, 64
HEAD_DIM = KV_LORA + ROPE
TOPK = 2048
SCALE = 1.0 / np.sqrt(128 + 64)
NEG = -1e30
INT_MIN = np.int32(-2147483648)

ST = 1024
NT = S // ST


def _idx_scores_body(pos_ref, qt_ref, kv_ref, sc_ref, idx_ref):
    b = pl.program_id(0)
    j = pl.program_id(1)
    kv_t = kv_ref[0]
    s = jax.lax.dot_general(kv_t, qt_ref[0], (((1,), (0,)), ((), ())),
                            preferred_element_type=jnp.float32)
    s = s * SCALE
    ids = jax.lax.broadcasted_iota(jnp.int32, (ST, 1), 0) + j * ST
    s = jnp.where(ids <= pos_ref[b], s, NEG)
    sc_ref[0] = s
    idx_ref[0] = jnp.transpose(jnp.mean(s, axis=1, keepdims=True))


def _idx_scores(q, kv, positions):
    qt = jnp.swapaxes(q, 1, 2)
    return pl.pallas_call(
        _idx_scores_body,
        grid=(B, NT),
        in_specs=[
            pl.BlockSpec(memory_space=pltpu.SMEM),
            pl.BlockSpec((1, HEAD_DIM, H), lambda b, j: (b, 0, 0)),
            pl.BlockSpec((1, ST, HEAD_DIM), lambda b, j: (b, j, 0)),
        ],
        out_specs=[
            pl.BlockSpec((1, ST, H), lambda b, j: (b, j, 0)),
            pl.BlockSpec((1, 1, ST), lambda b, j: (b, 0, j)),
        ],
        out_shape=[
            jax.ShapeDtypeStruct((B, S, H), jnp.float32),
            jax.ShapeDtypeStruct((B, 1, S), jnp.float32),
        ],
    )(positions, qt, kv)


_NVEC = S // 16


def _sc_threshold_body(idx_hbm, out_hbm, val_v, key_v, res_v):
    wid = lax.axis_index("s") * 2 + lax.axis_index("c")

    @pl.when(wid < B)
    def _():
        pltpu.sync_copy(idx_hbm.at[wid], val_v)

        def conv(i, carry):
            for u in range(8):
                sl = pl.ds((i * 8 + u) * 16, 16)
                bb = val_v[sl]
                key_v[sl] = jnp.where(
                    bb < 0, jnp.bitwise_xor(jnp.bitwise_not(bb), INT_MIN), bb)
            return carry
        lax.fori_loop(0, _NVEC // 8, conv, jnp.int32(0))

        def count_ge(t):
            def cstep(i, acc):
                for u in range(8):
                    kk = key_v[pl.ds((i * 8 + u) * 16, 16)]
                    acc = acc + jnp.where(kk >= t, 1, 0).astype(jnp.int32)
                return acc
            acc = lax.fori_loop(0, _NVEC // 8, cstep,
                                jnp.zeros((16,), jnp.int32))
            total = jnp.int32(0)
            for lane in range(16):
                total = total + acc[lane]
            return total

        p0 = jnp.where(count_ge(jnp.int32(0)) >= TOPK,
                       jnp.int32(0), jnp.int32(INT_MIN))

        def bstep(jj, p):
            bit = jnp.int32(30) - jj
            t = jnp.bitwise_or(p, jnp.left_shift(jnp.int32(1), bit))
            return jnp.where(count_ge(t) >= TOPK, t, p)
        p = lax.fori_loop(0, 31, bstep, p0)

        fb = jnp.where(p >= 0, p,
                       jnp.bitwise_not(jnp.bitwise_xor(p, INT_MIN)))
        res_v[...] = jnp.full((16,), fb, jnp.int32)
        pltpu.sync_copy(res_v, out_hbm.at[wid])


def _thresholds(idx_scores):
    mesh = plsc.VectorSubcoreMesh(core_axis_name="c", subcore_axis_name="s")
    fn = functools.partial(
        pl.kernel,
        out_type=jax.ShapeDtypeStruct((B, 16), jnp.int32),
        mesh=mesh,
        scratch_types=[
            pltpu.VMEM((S,), jnp.int32),
            pltpu.VMEM((S,), jnp.int32),
            pltpu.VMEM((16,), jnp.int32),
        ],
    )(_sc_threshold_body)
    bits = jax.lax.bitcast_convert_type(idx_scores.reshape(B, S), jnp.int32)
    th = jax.lax.bitcast_convert_type(fn(bits)[:, 0], jnp.float32)
    return jnp.maximum(th, -1e28)


def _flash_body(th_ref, sc_ref, kv_ref, idx_ref, sink_ref, out_ref,
                m_ref, l_ref, acc_ref):
    b = pl.program_id(0)
    j = pl.program_id(1)

    @pl.when(j == 0)
    def _init():
        m_ref[...] = sink_ref[0][:, None]
        l_ref[...] = jnp.ones((H, 1), jnp.float32)
        acc_ref[...] = jnp.zeros((H, KV_LORA), jnp.float32)

    s = jnp.transpose(sc_ref[0])
    sel = idx_ref[0] >= th_ref[b]
    s = jnp.where(sel, s, NEG)

    m_old = m_ref[...]
    m_new = jnp.maximum(m_old, jnp.max(s, axis=1, keepdims=True))
    alpha = jnp.exp(m_old - m_new)
    pr = jnp.exp(s - m_new)
    l_ref[...] = l_ref[...] * alpha + jnp.sum(pr, axis=1, keepdims=True)
    ckv = kv_ref[0][:, :KV_LORA].astype(jnp.bfloat16)
    pv = jax.lax.dot_general(pr.astype(jnp.bfloat16), ckv,
                             (((1,), (0,)), ((), ())),
                             preferred_element_type=jnp.float32)
    acc_ref[...] = acc_ref[...] * alpha + pv
    m_ref[...] = m_new

    @pl.when(j == NT - 1)
    def _fin():
        out_ref[0] = acc_ref[...] / l_ref[...]


def _flash(scores, kv, idx_scores, thresholds, attn_sink):
    return pl.pallas_call(
        _flash_body,
        grid=(B, NT),
        in_specs=[
            pl.BlockSpec(memory_space=pltpu.SMEM),
            pl.BlockSpec((1, ST, H), lambda b, j: (b, j, 0)),
            pl.BlockSpec((1, ST, HEAD_DIM), lambda b, j: (b, j, 0)),
            pl.BlockSpec((1, 1, ST), lambda b, j: (b, 0, j)),
            pl.BlockSpec((1, H), lambda b, j: (0, 0)),
        ],
        out_specs=pl.BlockSpec((1, H, KV_LORA), lambda b, j: (b, 0, 0)),
        out_shape=jax.ShapeDtypeStruct((B, H, KV_LORA), jnp.float32),
        scratch_shapes=[
            pltpu.VMEM((H, 1), jnp.float32),
            pltpu.VMEM((H, 1), jnp.float32),
            pltpu.VMEM((H, KV_LORA), jnp.float32),
        ],
    )(thresholds, scores, kv, idx_scores, attn_sink.reshape(1, H))


def kernel(q, kv, positions, attn_sink):
    scores, idx_scores = _idx_scores(q, kv, positions)
    th = _thresholds(idx_scores)
    return _flash(scores, kv, idx_scores, th, attn_sink)


# speedup vs baseline: 1.1492x; 1.1492x over previous
import functools

import jax
import jax.numpy as jnp
import numpy as np
from jax import lax
from jax.experimental import pallas as pl
from jax.experimental.pallas import tpu as pltpu
from jax.experimental.pallas import tpu_sc as plsc

B, H, S = 16, 64, 4096
KV_LORA, ROPE = 512, 64
HEAD_DIM = KV_LORA + ROPE
TOPK = 2048
SCALE = 1.0 / np.sqrt(128 + 64)
NEG = -1e30
INT_MIN = np.int32(-2147483648)

ST = 1024
NT = S // ST


def _clamp(b, j, pos):
    return jnp.minimum(j, pos[b] // ST)


def _idx_scores_body(pos_ref, q_ref, kv_ref, sc_ref, idx_ref):
    b = pl.program_id(0)
    j = pl.program_id(1)
    valid = j * ST <= pos_ref[b]

    @pl.when(valid)
    def _():
        kv_t = kv_ref[0]
        s = jax.lax.dot_general(q_ref[0], kv_t, (((1,), (1,)), ((), ())),
                                preferred_element_type=jnp.float32)
        s = s * SCALE
        ids = jax.lax.broadcasted_iota(jnp.int32, (1, ST), 1) + j * ST
        s = jnp.where(ids <= pos_ref[b], s, NEG)
        sc_ref[0] = s.astype(jnp.bfloat16)
        idx_ref[0] = jnp.mean(s, axis=0, keepdims=True)

    @pl.when(jnp.logical_not(valid))
    def _():
        idx_ref[0] = jnp.full((1, ST), NEG, jnp.float32)


def _idx_scores(q, kv, positions):
    return pl.pallas_call(
        _idx_scores_body,
        grid_spec=pltpu.PrefetchScalarGridSpec(
            num_scalar_prefetch=1,
            grid=(B, NT),
            in_specs=[
                pl.BlockSpec((1, H, HEAD_DIM), lambda b, j, pos: (b, 0, 0)),
                pl.BlockSpec((1, ST, HEAD_DIM),
                             lambda b, j, pos: (b, _clamp(b, j, pos), 0)),
            ],
            out_specs=[
                pl.BlockSpec((1, H, ST),
                             lambda b, j, pos: (b, 0, _clamp(b, j, pos))),
                pl.BlockSpec((1, 1, ST), lambda b, j, pos: (b, 0, j)),
            ],
        ),
        out_shape=[
            jax.ShapeDtypeStruct((B, H, S), jnp.bfloat16),
            jax.ShapeDtypeStruct((B, 1, S), jnp.float32),
        ],
    )(positions, q, kv)


_NVEC = S // 16


def _sc_threshold_body(idx_hbm, out_hbm, val_v, key_v, res_v):
    wid = lax.axis_index("s") * 2 + lax.axis_index("c")

    @pl.when(wid < B)
    def _():
        pltpu.sync_copy(idx_hbm.at[wid], val_v)

        def conv(i, carry):
            for u in range(8):
                sl = pl.ds((i * 8 + u) * 16, 16)
                bb = val_v[sl]
                key_v[sl] = jnp.where(
                    bb < 0, jnp.bitwise_xor(jnp.bitwise_not(bb), INT_MIN), bb)
            return carry
        lax.fori_loop(0, _NVEC // 8, conv, jnp.int32(0))

        def count_ge(t):
            def cstep(i, acc):
                for u in range(8):
                    kk = key_v[pl.ds((i * 8 + u) * 16, 16)]
                    acc = acc + jnp.where(kk >= t, 1, 0).astype(jnp.int32)
                return acc
            acc = lax.fori_loop(0, _NVEC // 8, cstep,
                                jnp.zeros((16,), jnp.int32))
            total = jnp.int32(0)
            for lane in range(16):
                total = total + acc[lane]
            return total

        p0 = jnp.where(count_ge(jnp.int32(0)) >= TOPK,
                       jnp.int32(0), jnp.int32(INT_MIN))

        def bstep(jj, p):
            bit = jnp.int32(30) - jj
            t = jnp.bitwise_or(p, jnp.left_shift(jnp.int32(1), bit))
            return jnp.where(count_ge(t) >= TOPK, t, p)
        p = lax.fori_loop(0, 31, bstep, p0)

        fb = jnp.where(p >= 0, p,
                       jnp.bitwise_not(jnp.bitwise_xor(p, INT_MIN)))
        res_v[...] = jnp.full((16,), fb, jnp.int32)
        pltpu.sync_copy(res_v, out_hbm.at[wid])


def _thresholds(idx_scores):
    mesh = plsc.VectorSubcoreMesh(core_axis_name="c", subcore_axis_name="s")
    fn = functools.partial(
        pl.kernel,
        out_type=jax.ShapeDtypeStruct((B, 16), jnp.int32),
        mesh=mesh,
        scratch_types=[
            pltpu.VMEM((S,), jnp.int32),
            pltpu.VMEM((S,), jnp.int32),
            pltpu.VMEM((16,), jnp.int32),
        ],
    )(_sc_threshold_body)
    bits = jax.lax.bitcast_convert_type(idx_scores.reshape(B, S), jnp.int32)
    th = jax.lax.bitcast_convert_type(fn(bits)[:, 0], jnp.float32)
    return jnp.maximum(th, -1e28)


def _flash_body(pos_ref, th_ref, sc_ref, kv_ref, idx_ref, sink_ref, out_ref,
                m_ref, l_ref, acc_ref):
    b = pl.program_id(0)
    j = pl.program_id(1)

    @pl.when(j == 0)
    def _init():
        m_ref[...] = sink_ref[0][:, None]
        l_ref[...] = jnp.ones((H, 1), jnp.float32)
        acc_ref[...] = jnp.zeros((H, KV_LORA), jnp.float32)

    @pl.when(j * ST <= pos_ref[b])
    def _():
        s = sc_ref[0].astype(jnp.float32)
        sel = idx_ref[0] >= th_ref[b]
        s = jnp.where(sel, s, NEG)

        m_old = m_ref[...]
        m_new = jnp.maximum(m_old, jnp.max(s, axis=1, keepdims=True))
        alpha = jnp.exp(m_old - m_new)
        pr = jnp.exp(s - m_new)
        l_ref[...] = l_ref[...] * alpha + jnp.sum(pr, axis=1, keepdims=True)
        ckv = kv_ref[0][:, :KV_LORA].astype(jnp.bfloat16)
        pv = jax.lax.dot_general(pr.astype(jnp.bfloat16), ckv,
                                 (((1,), (0,)), ((), ())),
                                 preferred_element_type=jnp.float32)
        acc_ref[...] = acc_ref[...] * alpha + pv
        m_ref[...] = m_new

    @pl.when(j == NT - 1)
    def _fin():
        out_ref[0] = acc_ref[...] / l_ref[...]


def _flash(scores, kv, idx_scores, thresholds, attn_sink, positions):
    return pl.pallas_call(
        _flash_body,
        grid_spec=pltpu.PrefetchScalarGridSpec(
            num_scalar_prefetch=1,
            grid=(B, NT),
            in_specs=[
                pl.BlockSpec(memory_space=pltpu.SMEM),
                pl.BlockSpec((1, H, ST),
                             lambda b, j, pos: (b, 0, _clamp(b, j, pos))),
                pl.BlockSpec((1, ST, HEAD_DIM),
                             lambda b, j, pos: (b, _clamp(b, j, pos), 0)),
                pl.BlockSpec((1, 1, ST), lambda b, j, pos: (b, 0, j)),
                pl.BlockSpec((1, H), lambda b, j, pos: (0, 0)),
            ],
            out_specs=pl.BlockSpec((1, H, KV_LORA),
                                   lambda b, j, pos: (b, 0, 0)),
            scratch_shapes=[
                pltpu.VMEM((H, 1), jnp.float32),
                pltpu.VMEM((H, 1), jnp.float32),
                pltpu.VMEM((H, KV_LORA), jnp.float32),
            ],
        ),
        out_shape=jax.ShapeDtypeStruct((B, H, KV_LORA), jnp.float32),
    )(positions, thresholds, scores, kv, idx_scores, attn_sink.reshape(1, H))


def kernel(q, kv, positions, attn_sink):
    scores, idx_scores = _idx_scores(q, kv, positions)
    th = _thresholds(idx_scores)
    return _flash(scores, kv, idx_scores, th, attn_sink, positions)
